# baseline (speedup 1.0000x reference)
import jax
import jax.numpy as jnp
from jax import lax
from jax.experimental import pallas as pl
from jax.experimental.pallas import tpu as pltpu
from jax.experimental.pallas import tpu_sc as plsc

B, C, H, W = 8, 16, 512, 512
P = H * W
NC, NS = 2, 16
NPT = P // NS
CP = 2048
NCH = NPT // CP
NV = CP // 16

PROWS = 516
PCOLS = 768
XOFF = 128
PLANE = PROWS * PCOLS
ZCHUNK = 24832
ZLAST = PLANE - (NS - 1) * ZCHUNK
ROWS_PT = H // NS

F32 = jnp.float32
I32 = jnp.int32


def _splat_body(frame, flow, out, plane,
                fxr, fyr, ir0,
                val0, val1,
                ua0, ua1, ua2, ua3, ub0, ub1, ub2, ub3,
                ja1, ja2, ja3, jb1, jb2, jb3,
                sv0, sv1, ssc0, ssc1, smisc):
    cid = lax.axis_index("c")
    sid = lax.axis_index("s")
    tile_px = sid * NPT
    lane_f = lax.iota(I32, 16).astype(F32)

    vals = (val0, val1)
    svs = (sv0, sv1)
    upds = ((ua0, ua1, ua2, ua3), (ub0, ub1, ub2, ub3))
    jdx = ((ja1, ja2, ja3), (jb1, jb2, jb3))
    sscs = (ssc0, ssc1)

    ZB = 2048
    zbuf = ua0

    def zero_plane():
        def zb_init(j, carry):
            zbuf[pl.ds(j * 16, 16)] = jnp.zeros((16,), F32)
            return carry

        lax.fori_loop(0, ZB // 16, zb_init, None, unroll=False)

        @pl.when(sid < NS - 1)
        def _():
            ds = [pltpu.async_copy(
                zbuf, plane.at[pl.ds(sid * ZCHUNK + z * ZB, ZB)], smisc)
                for z in range(12)]
            ds.append(pltpu.async_copy(
                zbuf.at[pl.ds(0, 256)],
                plane.at[pl.ds(sid * ZCHUNK + 12 * ZB, 256)], smisc))
            for d in ds:
                d.wait()

        @pl.when(sid == NS - 1)
        def _():
            ds = [pltpu.async_copy(
                zbuf, plane.at[pl.ds((NS - 1) * ZCHUNK + z * ZB, ZB)], smisc)
                for z in range(11)]
            ds.append(pltpu.async_copy(
                zbuf.at[pl.ds(0, ZLAST - 11 * ZB)],
                plane.at[pl.ds((NS - 1) * ZCHUNK + 11 * ZB, ZLAST - 11 * ZB)],
                smisc))
            for d in ds:
                d.wait()

    zero_plane()
    plsc.subcore_barrier()

    def one_image(ib, carry):
        b = cid * (B // NC) + ib

        for ch in range(NCH):
            pstart = tile_px + ch * CP
            r0 = pstart // W
            ds = []
            for r in range(CP // W):
                ds.append(pltpu.async_copy(
                    flow.at[b, 0, r0 + r], ua0.at[pl.ds(r * W, W)], smisc))
                ds.append(pltpu.async_copy(
                    flow.at[b, 1, r0 + r], ua1.at[pl.ds(r * W, W)], smisc))
            for d in ds:
                d.wait()

            def vreg(j, carry, _ch=ch, _pstart=pstart):
                p0 = _pstart + j * 16
                o = _ch * CP + j * 16
                xs = (p0 % W).astype(F32) + lane_f
                ys = jnp.full((16,), 1.0, F32) * (p0 // W).astype(F32)
                tx = jnp.clip(xs + ua0[pl.ds(j * 16, 16)], -3.0, 513.0)
                ty = jnp.clip(ys + ua1[pl.ds(j * 16, 16)], -3.0, 513.0)
                x0 = tx.astype(I32)
                y0 = ty.astype(I32)
                x0 = jnp.where(x0.astype(F32) > tx, x0 - 1, x0)
                y0 = jnp.where(y0.astype(F32) > ty, y0 - 1, y0)
                fxr[pl.ds(o, 16)] = tx - x0.astype(F32)
                fyr[pl.ds(o, 16)] = ty - y0.astype(F32)
                x0c = jnp.clip(x0, -2, 512)
                y0c = jnp.clip(y0, -2, 512)
                base = (y0c + 2) * PCOLS + (x0c + XOFF)
                ir0[pl.ds(o, 16)] = base
                return carry

            lax.fori_loop(0, NV, vreg, None, unroll=4)

        def one_channel(c, carry):
            vdesc = [None, None]
            scdesc = [[], []]
            r00 = tile_px // W
            vdesc[0] = [pltpu.async_copy(
                frame.at[b, c, r00 + r], val0.at[pl.ds(r * W, W)], sv0)
                for r in range(CP // W)]
            for ch in range(NCH):
                s = ch & 1
                for d in scdesc[s]:
                    d.wait()
                scdesc[s] = []
                for d in vdesc[s]:
                    d.wait()
                if ch < NCH - 1:
                    r1 = (tile_px + (ch + 1) * CP) // W
                    vdesc[1 - s] = [pltpu.async_copy(
                        frame.at[b, c, r1 + r], vals[1 - s].at[pl.ds(r * W, W)],
                        svs[1 - s]) for r in range(CP // W)]
                vb = vals[s]
                u0, u1, u2, u3 = upds[s]
                j1, j2, j3 = jdx[s]

                def vreg(j, carry, _ch=ch, _vb=vb,
                         _u0=u0, _u1=u1, _u2=u2, _u3=u3,
                         _j1=j1, _j2=j2, _j3=j3):
                    o = _ch * CP + j * 16
                    cc = j * 16
                    v = _vb[pl.ds(cc, 16)]
                    fx = fxr[pl.ds(o, 16)]
                    fy = fyr[pl.ds(o, 16)]
                    bs = ir0[pl.ds(o, 16)]
                    vgy = v - v * fy
                    vfy = v * fy
                    _u0[pl.ds(cc, 16)] = vgy - vgy * fx
                    _u1[pl.ds(cc, 16)] = vgy * fx
                    _u2[pl.ds(cc, 16)] = vfy - vfy * fx
                    _u3[pl.ds(cc, 16)] = vfy * fx
                    _j1[pl.ds(cc, 16)] = bs + 1
                    _j2[pl.ds(cc, 16)] = bs + PCOLS
                    _j3[pl.ds(cc, 16)] = bs + (PCOLS + 1)
                    return carry

                lax.fori_loop(0, NV, vreg, None, unroll=4)
                off = ch * CP
                scdesc[s].append(pltpu.async_copy(
                    upds[s][0], plane.at[ir0.at[pl.ds(off, CP)]],
                    sscs[s], add=True))
                for k in range(3):
                    scdesc[s].append(pltpu.async_copy(
                        upds[s][k + 1], plane.at[jdx[s][k]],
                        sscs[s], add=True))
            for s in range(2):
                for d in scdesc[s]:
                    d.wait()
            plsc.subcore_barrier()

            ddescs = []
            for r in range(ROWS_PT):
                row = sid * ROWS_PT + r
                src_off = pl.multiple_of((row + 2) * PCOLS + XOFF, 128)
                ddescs.append(pltpu.async_copy(
                    plane.at[pl.ds(src_off, W)],
                    out.at[b, c, row], smisc))
            for d in ddescs:
                d.wait()
            plsc.subcore_barrier()

            zero_plane()
            plsc.subcore_barrier()
            return carry

        lax.fori_loop(0, C, one_channel, None, unroll=False)
        return carry

    lax.fori_loop(0, B // NC, one_image, None, unroll=False)


def kernel(frame, flow):
    mesh = plsc.VectorSubcoreMesh(core_axis_name="c", subcore_axis_name="s")
    fn = pl.kernel(
        _splat_body,
        out_type=jax.ShapeDtypeStruct((B, C, H, W), F32),
        mesh=mesh,
        scratch_types=[
            pltpu.VMEM_SHARED((PLANE,), F32),
            pltpu.VMEM((NPT,), F32),
            pltpu.VMEM((NPT,), F32),
            pltpu.VMEM((NPT,), I32),
            pltpu.VMEM((CP,), F32),
            pltpu.VMEM((CP,), F32),
            pltpu.VMEM((CP,), F32),
            pltpu.VMEM((CP,), F32),
            pltpu.VMEM((CP,), F32),
            pltpu.VMEM((CP,), F32),
            pltpu.VMEM((CP,), F32),
            pltpu.VMEM((CP,), F32),
            pltpu.VMEM((CP,), F32),
            pltpu.VMEM((CP,), F32),
            pltpu.VMEM((CP,), I32),
            pltpu.VMEM((CP,), I32),
            pltpu.VMEM((CP,), I32),
            pltpu.VMEM((CP,), I32),
            pltpu.VMEM((CP,), I32),
            pltpu.VMEM((CP,), I32),
            pltpu.SemaphoreType.DMA,
            pltpu.SemaphoreType.DMA,
            pltpu.SemaphoreType.DMA,
            pltpu.SemaphoreType.DMA,
            pltpu.SemaphoreType.DMA,
        ],
    )
    return fn(frame, flow)


# speedup vs baseline: 1.0069x; 1.0069x over previous
import jax
import jax.numpy as jnp
from jax import lax
from jax.experimental import pallas as pl
from jax.experimental.pallas import tpu as pltpu
from jax.experimental.pallas import tpu_sc as plsc

B, C, H, W = 8, 16, 512, 512
P = H * W
NC, NS = 2, 16
NPT = P // NS
CP = 2048
NCH = NPT // CP
NV = CP // 16

PROWS = 516
PCOLS = 768
XOFF = 128
PLANE = PROWS * PCOLS
ZCHUNK = 24832
ZLAST = PLANE - (NS - 1) * ZCHUNK
ROWS_PT = H // NS

F32 = jnp.float32
I32 = jnp.int32


def _splat_body(frame, flow, out, plane,
                fxr, fyr, ir0,
                val0, val1,
                ua0, ua1, ua2, ua3, ub0, ub1, ub2, ub3,
                ja1, ja2, ja3, jb1, jb2, jb3,
                sv0, sv1, ssc0, ssc1, smisc):
    cid = lax.axis_index("c")
    sid = lax.axis_index("s")
    tile_px = sid * NPT
    lane_f = lax.iota(I32, 16).astype(F32)

    vals = (val0, val1)
    svs = (sv0, sv1)
    upds = ((ua0, ua1, ua2, ua3), (ub0, ub1, ub2, ub3))
    jdx = ((ja1, ja2, ja3), (jb1, jb2, jb3))
    sscs = (ssc0, ssc1)

    ZB = 2048
    zbuf = ua0

    def zero_plane():
        def zb_init(j, carry):
            zbuf[pl.ds(j * 16, 16)] = jnp.zeros((16,), F32)
            return carry

        lax.fori_loop(0, ZB // 16, zb_init, None, unroll=False)

        @pl.when(sid < NS - 1)
        def _():
            ds = [pltpu.async_copy(
                zbuf, plane.at[pl.ds(sid * ZCHUNK + z * ZB, ZB)], smisc)
                for z in range(12)]
            ds.append(pltpu.async_copy(
                zbuf.at[pl.ds(0, 256)],
                plane.at[pl.ds(sid * ZCHUNK + 12 * ZB, 256)], smisc))
            for d in ds:
                d.wait()

        @pl.when(sid == NS - 1)
        def _():
            ds = [pltpu.async_copy(
                zbuf, plane.at[pl.ds((NS - 1) * ZCHUNK + z * ZB, ZB)], smisc)
                for z in range(11)]
            ds.append(pltpu.async_copy(
                zbuf.at[pl.ds(0, ZLAST - 11 * ZB)],
                plane.at[pl.ds((NS - 1) * ZCHUNK + 11 * ZB, ZLAST - 11 * ZB)],
                smisc))
            for d in ds:
                d.wait()

    zero_plane()
    plsc.subcore_barrier()

    def one_image(ib, carry):
        b = cid * (B // NC) + ib

        for ch in range(NCH):
            pstart = tile_px + ch * CP
            r0 = pstart // W
            ds = []
            for r in range(CP // W):
                ds.append(pltpu.async_copy(
                    flow.at[b, 0, r0 + r], ua0.at[pl.ds(r * W, W)], smisc))
                ds.append(pltpu.async_copy(
                    flow.at[b, 1, r0 + r], ua1.at[pl.ds(r * W, W)], smisc))
            for d in ds:
                d.wait()

            def vreg(j, carry, _ch=ch, _pstart=pstart):
                p0 = _pstart + j * 16
                o = _ch * CP + j * 16
                xs = (p0 % W).astype(F32) + lane_f
                ys = jnp.full((16,), 1.0, F32) * (p0 // W).astype(F32)
                tx = jnp.clip(xs + ua0[pl.ds(j * 16, 16)], -3.0, 513.0)
                ty = jnp.clip(ys + ua1[pl.ds(j * 16, 16)], -3.0, 513.0)
                x0 = tx.astype(I32)
                y0 = ty.astype(I32)
                x0 = jnp.where(x0.astype(F32) > tx, x0 - 1, x0)
                y0 = jnp.where(y0.astype(F32) > ty, y0 - 1, y0)
                fxr[pl.ds(o, 16)] = tx - x0.astype(F32)
                fyr[pl.ds(o, 16)] = ty - y0.astype(F32)
                x0c = jnp.clip(x0, -2, 512)
                y0c = jnp.clip(y0, -2, 512)
                base = (y0c + 2) * PCOLS + (x0c + XOFF)
                ir0[pl.ds(o, 16)] = base
                return carry

            lax.fori_loop(0, NV, vreg, None, unroll=False)

        def one_channel(c, carry):
            vdesc = [None, None]
            scdesc = [[], []]
            r00 = tile_px // W
            vdesc[0] = [pltpu.async_copy(
                frame.at[b, c, r00 + r], val0.at[pl.ds(r * W, W)], sv0)
                for r in range(CP // W)]
            for ch in range(NCH):
                s = ch & 1
                for d in scdesc[s]:
                    d.wait()
                scdesc[s] = []
                for d in vdesc[s]:
                    d.wait()
                if ch < NCH - 1:
                    r1 = (tile_px + (ch + 1) * CP) // W
                    vdesc[1 - s] = [pltpu.async_copy(
                        frame.at[b, c, r1 + r], vals[1 - s].at[pl.ds(r * W, W)],
                        svs[1 - s]) for r in range(CP // W)]
                vb = vals[s]
                u0, u1, u2, u3 = upds[s]
                j1, j2, j3 = jdx[s]

                def vreg(j, carry, _ch=ch, _vb=vb,
                         _u0=u0, _u1=u1, _u2=u2, _u3=u3,
                         _j1=j1, _j2=j2, _j3=j3):
                    o = _ch * CP + j * 16
                    cc = j * 16
                    v = _vb[pl.ds(cc, 16)]
                    fx = fxr[pl.ds(o, 16)]
                    fy = fyr[pl.ds(o, 16)]
                    bs = ir0[pl.ds(o, 16)]
                    vgy = v - v * fy
                    vfy = v * fy
                    _u0[pl.ds(cc, 16)] = vgy - vgy * fx
                    _u1[pl.ds(cc, 16)] = vgy * fx
                    _u2[pl.ds(cc, 16)] = vfy - vfy * fx
                    _u3[pl.ds(cc, 16)] = vfy * fx
                    _j1[pl.ds(cc, 16)] = bs + 1
                    _j2[pl.ds(cc, 16)] = bs + PCOLS
                    _j3[pl.ds(cc, 16)] = bs + (PCOLS + 1)
                    return carry

                lax.fori_loop(0, NV, vreg, None, unroll=False)
                off = ch * CP
                scdesc[s].append(pltpu.async_copy(
                    upds[s][0], plane.at[ir0.at[pl.ds(off, CP)]],
                    sscs[s], add=True))
                for k in range(3):
                    scdesc[s].append(pltpu.async_copy(
                        upds[s][k + 1], plane.at[jdx[s][k]],
                        sscs[s], add=True))
            for s in range(2):
                for d in scdesc[s]:
                    d.wait()
            plsc.subcore_barrier()

            ddescs = []
            for r in range(ROWS_PT):
                row = sid * ROWS_PT + r
                src_off = pl.multiple_of((row + 2) * PCOLS + XOFF, 128)
                ddescs.append(pltpu.async_copy(
                    plane.at[pl.ds(src_off, W)],
                    out.at[b, c, row], smisc))
            for d in ddescs:
                d.wait()
            plsc.subcore_barrier()

            zero_plane()
            plsc.subcore_barrier()
            return carry

        lax.fori_loop(0, C, one_channel, None, unroll=False)
        return carry

    lax.fori_loop(0, B // NC, one_image, None, unroll=False)


def kernel(frame, flow):
    mesh = plsc.VectorSubcoreMesh(core_axis_name="c", subcore_axis_name="s")
    fn = pl.kernel(
        _splat_body,
        out_type=jax.ShapeDtypeStruct((B, C, H, W), F32),
        mesh=mesh,
        scratch_types=[
            pltpu.VMEM_SHARED((PLANE,), F32),
            pltpu.VMEM((NPT,), F32),
            pltpu.VMEM((NPT,), F32),
            pltpu.VMEM((NPT,), I32),
            pltpu.VMEM((CP,), F32),
            pltpu.VMEM((CP,), F32),
            pltpu.VMEM((CP,), F32),
            pltpu.VMEM((CP,), F32),
            pltpu.VMEM((CP,), F32),
            pltpu.VMEM((CP,), F32),
            pltpu.VMEM((CP,), F32),
            pltpu.VMEM((CP,), F32),
            pltpu.VMEM((CP,), F32),
            pltpu.VMEM((CP,), F32),
            pltpu.VMEM((CP,), I32),
            pltpu.VMEM((CP,), I32),
            pltpu.VMEM((CP,), I32),
            pltpu.VMEM((CP,), I32),
            pltpu.VMEM((CP,), I32),
            pltpu.VMEM((CP,), I32),
            pltpu.SemaphoreType.DMA,
            pltpu.SemaphoreType.DMA,
            pltpu.SemaphoreType.DMA,
            pltpu.SemaphoreType.DMA,
            pltpu.SemaphoreType.DMA,
        ],
    )
    return fn(frame, flow)
